# baseline (speedup 1.0000x reference)
import functools

import jax
import jax.numpy as jnp
from jax import lax
from jax.experimental import pallas as pl
from jax.experimental.pallas import tpu as pltpu
from jax.experimental.pallas import tpu_sc as plsc

L = 32
N = 32768
LANES = 16
NSTREAM = 8
SSZ = N // NSTREAM
SCH = SSZ // LANES
BITS = (11, 11, 10)
SHIFTS = (0, 11, 22)
NB = 2048


def _i32(x):
    return plsc.bitcast(x, jnp.int32)


def _f32(x):
    return plsc.bitcast(x, jnp.float32)


def _sort_scatter_body(km_hbm, fn_hbm, out_hbm, keys, pay_a, pay_b, *hists):
    wid = lax.axis_index("c") * 16 + lax.axis_index("s")
    lane = lax.iota(jnp.int32, 16)

    pltpu.sync_copy(fn_hbm.at[wid], keys)

    for p in range(3):
        shift = SHIFTS[p]
        mask = (1 << BITS[p]) - 1

        @plsc.parallel_loop(0, (mask + 1) // 16, unroll=8)
        def _zero(v):
            for h in hists:
                h[pl.ds(v * 16, 16)] = jnp.zeros((16,), jnp.int32)

        def _count(i, carry, p=p, shift=shift, mask=mask):
            if p == 0:
                js = [16 * i + lane + u * SSZ for u in range(NSTREAM)]
                ks = []
                for u in range(NSTREAM):
                    b = _i32(keys[pl.ds((i + u * SCH) * 16, 16)])
                    m = jnp.where(b >= 0, b ^ 0x7FFFFFFF, b)
                    keys[pl.ds((i + u * SCH) * 16, 16)] = _f32(m)
                    ks.append(m)
            else:
                js = [pay_a[pl.ds((i + u * SCH) * 16, 16)] & (N - 1)
                      for u in range(NSTREAM)]
                ks = [_i32(plsc.load_gather(keys, [j])) for j in js]
            ds = [lax.shift_right_logical(k, shift) & mask for k in ks]
            sc = [plsc.scan_count(d) for d in ds]
            cs = [plsc.load_gather(hists[u], [ds[u]])
                  for u in range(NSTREAM)]
            tots = [cs[u] + sc[u][0] for u in range(NSTREAM)]
            occs = [t - 1 for t in tots]
            for u in range(NSTREAM):
                plsc.store_scatter(hists[u], [ds[u]], tots[u],
                                   mask=sc[u][1])
            for u in range(NSTREAM):
                pay_b[pl.ds((i + u * SCH) * 16, 16)] = _f32(
                    (occs[u] << 15) | js[u])
            return carry

        lax.fori_loop(0, SCH, _count, jnp.int32(0))

        def _prefix(v, carry):
            xs = [hists[u][pl.ds(v * 16, 16)] for u in range(NSTREAM)]
            s = xs[0]
            for u in range(1, NSTREAM):
                s = s + xs[u]
            base = plsc.cumsum(s) - s + carry
            for u in range(NSTREAM):
                hists[u][pl.ds(v * 16, 16)] = base
                base = base + xs[u]
            return carry + jnp.sum(s)

        lax.fori_loop(0, (mask + 1) // 16, _prefix, jnp.int32(0))

        @plsc.parallel_loop(0, SCH, unroll=4)
        def _permute(i, shift=shift, mask=mask):
            ws = [_i32(pay_b[pl.ds((i + u * SCH) * 16, 16)])
                  for u in range(NSTREAM)]
            js = [w & 0x7FFF for w in ws]
            occs = [lax.shift_right_logical(w, 15) for w in ws]
            ks = [_i32(plsc.load_gather(keys, [j])) for j in js]
            bases = [
                plsc.load_gather(
                    hists[u],
                    [lax.shift_right_logical(ks[u], shift) & mask])
                for u in range(NSTREAM)
            ]
            for u in range(NSTREAM):
                plsc.store_scatter(pay_a, [(bases[u] + occs[u]) & (N - 1)],
                                   js[u])

    pltpu.sync_copy(km_hbm.at[wid], pay_b)

    @plsc.parallel_loop(0, N // 16, unroll=8)
    def _scatter(i):
        j = pay_a[pl.ds(i * 16, 16)] & (N - 1)
        v = pay_b[pl.ds(i * 16, 16)]
        plsc.store_scatter(keys, [j], v)

    pltpu.sync_copy(keys, out_hbm.at[wid])


@jax.jit
def _sc_sort_scatter(km, fn):
    mesh = plsc.VectorSubcoreMesh(core_axis_name="c", subcore_axis_name="s")
    f = pl.kernel(
        _sort_scatter_body,
        out_type=jax.ShapeDtypeStruct((L, N), jnp.float32),
        mesh=mesh,
        compiler_params=pltpu.CompilerParams(needs_layout_passes=False),
        scratch_types=[
            pltpu.VMEM((N,), jnp.float32),
            pltpu.VMEM((N,), jnp.int32),
            pltpu.VMEM((N,), jnp.float32),
        ] + [pltpu.VMEM((NB,), jnp.int32)] * NSTREAM,
    )
    return f(km, fn)


def _sigmoid_body(x_ref, m_ref, o_ref):
    t = (x_ref[...] - m_ref[0, 0]) / 0.7
    o_ref[...] = 1.0 / (1.0 + jnp.exp(-t))


@jax.jit
def _tc_sigmoid(fn, mean):
    return pl.pallas_call(
        _sigmoid_body,
        out_shape=jax.ShapeDtypeStruct((L, N), jnp.float32),
        in_specs=[
            pl.BlockSpec((L, N), lambda: (0, 0)),
            pl.BlockSpec(memory_space=pltpu.SMEM),
        ],
        out_specs=pl.BlockSpec((L, N), lambda: (0, 0)),
    )(fn, jnp.reshape(mean, (1, 1)))


def kernel(k_masks, fn, mean):
    ori_masks = _sc_sort_scatter(k_masks, fn)
    cout = _tc_sigmoid(fn, mean)
    return ori_masks, cout


# speedup vs baseline: 1.0598x; 1.0598x over previous
import functools

import jax
import jax.numpy as jnp
from jax import lax
from jax.experimental import pallas as pl
from jax.experimental.pallas import tpu as pltpu
from jax.experimental.pallas import tpu_sc as plsc

L = 32
N = 32768
LANES = 16
NSTREAM = 8
SSZ = N // NSTREAM
SCH = SSZ // LANES
BITS = (11, 11, 10)
SHIFTS = (0, 11, 22)
NB = 2048


def _i32(x):
    return plsc.bitcast(x, jnp.int32)


def _f32(x):
    return plsc.bitcast(x, jnp.float32)


def _sort_scatter_body(km_hbm, fn_hbm, out_hbm, keys, pay_a, pay_b, *hists):
    wid = lax.axis_index("c") * 16 + lax.axis_index("s")
    lane = lax.iota(jnp.int32, 16)

    pltpu.sync_copy(fn_hbm.at[wid], keys)

    for p in range(3):
        shift = SHIFTS[p]
        mask = (1 << BITS[p]) - 1

        @plsc.parallel_loop(0, (mask + 1) // 16, unroll=8)
        def _zero(v):
            for h in hists:
                h[pl.ds(v * 16, 16)] = jnp.zeros((16,), jnp.int32)

        def _count(i2, carry, p=p, shift=shift, mask=mask):
            js, ds, sc = [], [], []
            for h in range(2):
                i = 2 * i2 + h
                if p == 0:
                    jh = [16 * i + lane + u * SSZ for u in range(NSTREAM)]
                    kh = []
                    for u in range(NSTREAM):
                        b = _i32(keys[pl.ds((i + u * SCH) * 16, 16)])
                        m = jnp.where(b >= 0, b ^ 0x7FFFFFFF, b)
                        keys[pl.ds((i + u * SCH) * 16, 16)] = _f32(m)
                        kh.append(m)
                else:
                    jh = [pay_a[pl.ds((i + u * SCH) * 16, 16)] & (N - 1)
                          for u in range(NSTREAM)]
                    kh = [_i32(plsc.load_gather(keys, [j])) for j in jh]
                dh = [lax.shift_right_logical(k, shift) & mask for k in kh]
                js.append(jh)
                ds.append(dh)
                sc.append([plsc.scan_count(d) for d in dh])
            for h in range(2):
                i = 2 * i2 + h
                cs = [plsc.load_gather(hists[u], [ds[h][u]])
                      for u in range(NSTREAM)]
                tots = [cs[u] + sc[h][u][0] for u in range(NSTREAM)]
                for u in range(NSTREAM):
                    plsc.store_scatter(hists[u], [ds[h][u]], tots[u],
                                       mask=sc[h][u][1])
                for u in range(NSTREAM):
                    pay_b[pl.ds((i + u * SCH) * 16, 16)] = _f32(
                        ((tots[u] - 1) << 15) | js[h][u])
            return carry

        lax.fori_loop(0, SCH // 2, _count, jnp.int32(0))

        def _prefix(v, carry):
            xs = [hists[u][pl.ds(v * 16, 16)] for u in range(NSTREAM)]
            s = xs[0]
            for u in range(1, NSTREAM):
                s = s + xs[u]
            base = plsc.cumsum(s) - s + carry
            for u in range(NSTREAM):
                hists[u][pl.ds(v * 16, 16)] = base
                base = base + xs[u]
            return carry + jnp.sum(s)

        lax.fori_loop(0, (mask + 1) // 16, _prefix, jnp.int32(0))

        @plsc.parallel_loop(0, SCH, unroll=2)
        def _permute(i, shift=shift, mask=mask):
            ws = [_i32(pay_b[pl.ds((i + u * SCH) * 16, 16)])
                  for u in range(NSTREAM)]
            js = [w & 0x7FFF for w in ws]
            occs = [lax.shift_right_logical(w, 15) for w in ws]
            ks = [_i32(plsc.load_gather(keys, [j])) for j in js]
            bases = [
                plsc.load_gather(
                    hists[u],
                    [lax.shift_right_logical(ks[u], shift) & mask])
                for u in range(NSTREAM)
            ]
            for u in range(NSTREAM):
                plsc.store_scatter(pay_a, [(bases[u] + occs[u]) & (N - 1)],
                                   js[u])

    pltpu.sync_copy(km_hbm.at[wid], pay_b)

    @plsc.parallel_loop(0, N // 16, unroll=8)
    def _scatter(i):
        j = pay_a[pl.ds(i * 16, 16)] & (N - 1)
        v = pay_b[pl.ds(i * 16, 16)]
        plsc.store_scatter(keys, [j], v)

    pltpu.sync_copy(keys, out_hbm.at[wid])


@jax.jit
def _sc_sort_scatter(km, fn):
    mesh = plsc.VectorSubcoreMesh(core_axis_name="c", subcore_axis_name="s")
    f = pl.kernel(
        _sort_scatter_body,
        out_type=jax.ShapeDtypeStruct((L, N), jnp.float32),
        mesh=mesh,
        compiler_params=pltpu.CompilerParams(needs_layout_passes=False),
        scratch_types=[
            pltpu.VMEM((N,), jnp.float32),
            pltpu.VMEM((N,), jnp.int32),
            pltpu.VMEM((N,), jnp.float32),
        ] + [pltpu.VMEM((NB,), jnp.int32)] * NSTREAM,
    )
    return f(km, fn)


def _sigmoid_body(x_ref, m_ref, o_ref):
    t = (x_ref[...] - m_ref[0, 0]) / 0.7
    o_ref[...] = 1.0 / (1.0 + jnp.exp(-t))


@jax.jit
def _tc_sigmoid(fn, mean):
    return pl.pallas_call(
        _sigmoid_body,
        out_shape=jax.ShapeDtypeStruct((L, N), jnp.float32),
        in_specs=[
            pl.BlockSpec((L, N), lambda: (0, 0)),
            pl.BlockSpec(memory_space=pltpu.SMEM),
        ],
        out_specs=pl.BlockSpec((L, N), lambda: (0, 0)),
    )(fn, jnp.reshape(mean, (1, 1)))


def kernel(k_masks, fn, mean):
    ori_masks = _sc_sort_scatter(k_masks, fn)
    cout = _tc_sigmoid(fn, mean)
    return ori_masks, cout


# speedup vs baseline: 1.0620x; 1.0021x over previous
import functools

import jax
import jax.numpy as jnp
from jax import lax
from jax.experimental import pallas as pl
from jax.experimental.pallas import tpu as pltpu
from jax.experimental.pallas import tpu_sc as plsc

L = 32
N = 32768
LANES = 16
NSTREAM = 8
SSZ = N // NSTREAM
SCH = SSZ // LANES
BITS = (11, 11, 10)
SHIFTS = (0, 11, 22)
NB = 2048


def _i32(x):
    return plsc.bitcast(x, jnp.int32)


def _f32(x):
    return plsc.bitcast(x, jnp.float32)


def _sort_scatter_body(km_hbm, fn_hbm, out_hbm, keys, pay_a, pay_b, *hists):
    wid = lax.axis_index("c") * 16 + lax.axis_index("s")
    lane = lax.iota(jnp.int32, 16)

    pltpu.sync_copy(fn_hbm.at[wid], keys)

    for p in range(3):
        shift = SHIFTS[p]
        mask = (1 << BITS[p]) - 1

        @plsc.parallel_loop(0, (mask + 1) // 16, unroll=8)
        def _zero(v):
            for h in hists:
                h[pl.ds(v * 16, 16)] = jnp.zeros((16,), jnp.int32)

        def _count(i2, carry, p=p, shift=shift, mask=mask):
            js, ds, sc = [], [], []
            for h in range(2):
                i = 2 * i2 + h
                if p == 0:
                    jh = [16 * i + lane + u * SSZ for u in range(NSTREAM)]
                    kh = []
                    for u in range(NSTREAM):
                        b = _i32(keys[pl.ds((i + u * SCH) * 16, 16)])
                        m = jnp.where(b >= 0, b ^ 0x7FFFFFFF, b)
                        keys[pl.ds((i + u * SCH) * 16, 16)] = _f32(m)
                        kh.append(m)
                else:
                    jh = [pay_a[pl.ds((i + u * SCH) * 16, 16)]
                          for u in range(NSTREAM)]
                    kh = [_i32(plsc.load_gather(keys, [j])) for j in jh]
                dh = [lax.shift_right_logical(k, shift) & mask for k in kh]
                js.append(jh)
                ds.append(dh)
                sc.append([plsc.scan_count(d) for d in dh])
            for h in range(2):
                i = 2 * i2 + h
                cs = [plsc.load_gather(hists[u], [ds[h][u]])
                      for u in range(NSTREAM)]
                tots = [cs[u] + sc[h][u][0] for u in range(NSTREAM)]
                for u in range(NSTREAM):
                    plsc.store_scatter(hists[u], [ds[h][u]], tots[u],
                                       mask=sc[h][u][1])
                for u in range(NSTREAM):
                    pay_b[pl.ds((i + u * SCH) * 16, 16)] = _f32(
                        ((tots[u] - 1) << 15) | js[h][u])
            return carry

        lax.fori_loop(0, SCH // 2, _count, jnp.int32(0))

        def _prefix(v, carry):
            xs = [hists[u][pl.ds(v * 16, 16)] for u in range(NSTREAM)]
            s = xs[0]
            for u in range(1, NSTREAM):
                s = s + xs[u]
            base = plsc.cumsum(s) - s + carry
            for u in range(NSTREAM):
                hists[u][pl.ds(v * 16, 16)] = base
                base = base + xs[u]
            return carry + jnp.sum(s)

        lax.fori_loop(0, (mask + 1) // 16, _prefix, jnp.int32(0))

        @plsc.parallel_loop(0, SCH, unroll=2)
        def _permute(i, shift=shift, mask=mask):
            ws = [_i32(pay_b[pl.ds((i + u * SCH) * 16, 16)])
                  for u in range(NSTREAM)]
            js = [w & 0x7FFF for w in ws]
            occs = [lax.shift_right_logical(w, 15) for w in ws]
            ks = [_i32(plsc.load_gather(keys, [j])) for j in js]
            bases = [
                plsc.load_gather(
                    hists[u],
                    [lax.shift_right_logical(ks[u], shift) & mask])
                for u in range(NSTREAM)
            ]
            for u in range(NSTREAM):
                plsc.store_scatter(pay_a, [bases[u] + occs[u]], js[u])

    pltpu.sync_copy(km_hbm.at[wid], pay_b)

    @plsc.parallel_loop(0, N // 16, unroll=8)
    def _scatter(i):
        j = pay_a[pl.ds(i * 16, 16)]
        v = pay_b[pl.ds(i * 16, 16)]
        plsc.store_scatter(keys, [j], v)

    pltpu.sync_copy(keys, out_hbm.at[wid])


@jax.jit
def _sc_sort_scatter(km, fn):
    mesh = plsc.VectorSubcoreMesh(core_axis_name="c", subcore_axis_name="s")
    f = pl.kernel(
        _sort_scatter_body,
        out_type=jax.ShapeDtypeStruct((L, N), jnp.float32),
        mesh=mesh,
        compiler_params=pltpu.CompilerParams(needs_layout_passes=False),
        scratch_types=[
            pltpu.VMEM((N,), jnp.float32),
            pltpu.VMEM((N,), jnp.int32),
            pltpu.VMEM((N,), jnp.float32),
        ] + [pltpu.VMEM((NB,), jnp.int32)] * NSTREAM,
    )
    return f(km, fn)


def _sigmoid_body(x_ref, m_ref, o_ref):
    t = (x_ref[...] - m_ref[0, 0]) / 0.7
    o_ref[...] = 1.0 / (1.0 + jnp.exp(-t))


@jax.jit
def _tc_sigmoid(fn, mean):
    return pl.pallas_call(
        _sigmoid_body,
        out_shape=jax.ShapeDtypeStruct((L, N), jnp.float32),
        in_specs=[
            pl.BlockSpec((L, N), lambda: (0, 0)),
            pl.BlockSpec(memory_space=pltpu.SMEM),
        ],
        out_specs=pl.BlockSpec((L, N), lambda: (0, 0)),
    )(fn, jnp.reshape(mean, (1, 1)))


def kernel(k_masks, fn, mean):
    ori_masks = _sc_sort_scatter(k_masks, fn)
    cout = _tc_sigmoid(fn, mean)
    return ori_masks, cout


# speedup vs baseline: 1.0789x; 1.0160x over previous
import functools

import jax
import jax.numpy as jnp
from jax import lax
from jax.experimental import pallas as pl
from jax.experimental.pallas import tpu as pltpu
from jax.experimental.pallas import tpu_sc as plsc

L = 32
N = 32768
LANES = 16
NSTREAM = 8
SSZ = N // NSTREAM
SCH = SSZ // LANES
BITS = (11, 11, 10)
SHIFTS = (0, 11, 22)
NB = 2048


def _i32(x):
    return plsc.bitcast(x, jnp.int32)


def _f32(x):
    return plsc.bitcast(x, jnp.float32)


def _sort_scatter_body(km_hbm, fn_hbm, out_hbm, keys, pay_a, pay_b, *hists):
    wid = lax.axis_index("c") * 16 + lax.axis_index("s")
    lane = lax.iota(jnp.int32, 16)

    pltpu.sync_copy(fn_hbm.at[wid], keys)

    for p in range(3):
        shift = SHIFTS[p]
        mask = (1 << BITS[p]) - 1

        @plsc.parallel_loop(0, (mask + 1) // 16, unroll=8)
        def _zero(v):
            for h in hists:
                h[pl.ds(v * 16, 16)] = jnp.zeros((16,), jnp.int32)

        def _count(i2, carry, p=p, shift=shift, mask=mask):
            js, ds, sc = [], [], []
            for h in range(2):
                i = 2 * i2 + h
                if p == 0:
                    jh = [16 * i + lane + u * SSZ for u in range(NSTREAM)]
                    kh = []
                    for u in range(NSTREAM):
                        b = _i32(keys[pl.ds((i + u * SCH) * 16, 16)])
                        m = jnp.where(b >= 0, b ^ 0x7FFFFFFF, b)
                        keys[pl.ds((i + u * SCH) * 16, 16)] = _f32(m)
                        kh.append(m)
                else:
                    jh = [pay_a[pl.ds((i + u * SCH) * 16, 16)]
                          for u in range(NSTREAM)]
                    kh = [_i32(plsc.load_gather(keys, [j])) for j in jh]
                dh = [lax.shift_right_logical(k, shift) & mask for k in kh]
                js.append(jh)
                ds.append(dh)
                sc.append([plsc.scan_count(d) for d in dh])
            for h in range(2):
                i = 2 * i2 + h
                cs = [plsc.load_gather(hists[u], [ds[h][u]])
                      for u in range(NSTREAM)]
                tots = [cs[u] + sc[h][u][0] for u in range(NSTREAM)]
                for u in range(NSTREAM):
                    plsc.store_scatter(hists[u], [ds[h][u]], tots[u],
                                       mask=sc[h][u][1])
                for u in range(NSTREAM):
                    pay_b[pl.ds((i + u * SCH) * 16, 16)] = _f32(
                        ((tots[u] - 1) << 15) | js[h][u])
            return carry

        lax.fori_loop(0, SCH // 2, _count, jnp.int32(0))

        def _prefix(v, carry):
            xs = [hists[u][pl.ds(v * 16, 16)] for u in range(NSTREAM)]
            s = xs[0]
            for u in range(1, NSTREAM):
                s = s + xs[u]
            base = plsc.cumsum(s) - s + carry
            for u in range(NSTREAM):
                hists[u][pl.ds(v * 16, 16)] = base
                base = base + xs[u]
            return carry + jnp.sum(s)

        lax.fori_loop(0, (mask + 1) // 16, _prefix, jnp.int32(0))

        @plsc.parallel_loop(0, SCH, step=2)
        def _permute(i0, shift=shift, mask=mask):
            iis = [(i0 + h, u) for h in range(2) for u in range(NSTREAM)]
            ws = [_i32(pay_b[pl.ds((i + u * SCH) * 16, 16)])
                  for i, u in iis]
            js = [w & 0x7FFF for w in ws]
            occs = [lax.shift_right_logical(w, 15) for w in ws]
            ks = [_i32(plsc.load_gather(keys, [j])) for j in js]
            bases = [
                plsc.load_gather(
                    hists[iis[x][1]],
                    [lax.shift_right_logical(ks[x], shift) & mask])
                for x in range(2 * NSTREAM)
            ]
            for x in range(2 * NSTREAM):
                plsc.store_scatter(pay_a, [bases[x] + occs[x]], js[x])

    pltpu.sync_copy(km_hbm.at[wid], pay_b)

    @plsc.parallel_loop(0, N // 16, unroll=8)
    def _scatter(i):
        j = pay_a[pl.ds(i * 16, 16)]
        v = pay_b[pl.ds(i * 16, 16)]
        plsc.store_scatter(keys, [j], v)

    pltpu.sync_copy(keys, out_hbm.at[wid])


@jax.jit
def _sc_sort_scatter(km, fn):
    mesh = plsc.VectorSubcoreMesh(core_axis_name="c", subcore_axis_name="s")
    f = pl.kernel(
        _sort_scatter_body,
        out_type=jax.ShapeDtypeStruct((L, N), jnp.float32),
        mesh=mesh,
        compiler_params=pltpu.CompilerParams(needs_layout_passes=False),
        scratch_types=[
            pltpu.VMEM((N,), jnp.float32),
            pltpu.VMEM((N,), jnp.int32),
            pltpu.VMEM((N,), jnp.float32),
        ] + [pltpu.VMEM((NB,), jnp.int32)] * NSTREAM,
    )
    return f(km, fn)


def _sigmoid_body(x_ref, m_ref, o_ref):
    t = (x_ref[...] - m_ref[0, 0]) / 0.7
    o_ref[...] = 1.0 / (1.0 + jnp.exp(-t))


@jax.jit
def _tc_sigmoid(fn, mean):
    return pl.pallas_call(
        _sigmoid_body,
        out_shape=jax.ShapeDtypeStruct((L, N), jnp.float32),
        in_specs=[
            pl.BlockSpec((L, N), lambda: (0, 0)),
            pl.BlockSpec(memory_space=pltpu.SMEM),
        ],
        out_specs=pl.BlockSpec((L, N), lambda: (0, 0)),
    )(fn, jnp.reshape(mean, (1, 1)))


def kernel(k_masks, fn, mean):
    ori_masks = _sc_sort_scatter(k_masks, fn)
    cout = _tc_sigmoid(fn, mean)
    return ori_masks, cout


# speedup vs baseline: 1.0791x; 1.0002x over previous
import jax
import jax.numpy as jnp
from jax import lax
from jax.experimental import pallas as pl
from jax.experimental.pallas import tpu as pltpu
from jax.experimental.pallas import tpu_sc as plsc

L = 32
N = 32768
LANES = 16
NSTREAM = 8
SSZ = N // NSTREAM
SCH = SSZ // LANES
BITS = (11, 11, 10)
SHIFTS = (0, 11, 22)
NB = 2048


def _i32(x):
    return plsc.bitcast(x, jnp.int32)


def _f32(x):
    return plsc.bitcast(x, jnp.float32)


def _sort_scatter_body(km_hbm, fn_hbm, out_hbm, keys, pay_a, pay_b, *hists):
    wid = lax.axis_index("c") * 16 + lax.axis_index("s")
    lane = lax.iota(jnp.int32, 16)

    pltpu.sync_copy(fn_hbm.at[wid], keys)

    for p in range(3):
        shift = SHIFTS[p]
        mask = (1 << BITS[p]) - 1

        @plsc.parallel_loop(0, (mask + 1) // 16, unroll=8)
        def _zero(v):
            for h in hists:
                h[pl.ds(v * 16, 16)] = jnp.zeros((16,), jnp.int32)

        def _count(i2, carry, p=p, shift=shift, mask=mask):
            js, ds, sc = [], [], []
            for h in range(2):
                i = 2 * i2 + h
                if p == 0:
                    jh = [16 * i + lane + u * SSZ for u in range(NSTREAM)]
                    kh = []
                    for u in range(NSTREAM):
                        b = _i32(keys[pl.ds((i + u * SCH) * 16, 16)])
                        m = jnp.where(b >= 0, b ^ 0x7FFFFFFF, b)
                        keys[pl.ds((i + u * SCH) * 16, 16)] = _f32(m)
                        kh.append(m)
                else:
                    jh = [pay_a[pl.ds((i + u * SCH) * 16, 16)]
                          for u in range(NSTREAM)]
                    kh = [_i32(plsc.load_gather(keys, [j])) for j in jh]
                dh = [lax.shift_right_logical(k, shift) & mask for k in kh]
                js.append(jh)
                ds.append(dh)
                sc.append([plsc.scan_count(d) for d in dh])
            for h in range(2):
                i = 2 * i2 + h
                cs = [plsc.load_gather(hists[u], [ds[h][u]])
                      for u in range(NSTREAM)]
                tots = [cs[u] + sc[h][u][0] for u in range(NSTREAM)]
                for u in range(NSTREAM):
                    plsc.store_scatter(hists[u], [ds[h][u]], tots[u],
                                       mask=sc[h][u][1])
                for u in range(NSTREAM):
                    pay_b[pl.ds((i + u * SCH) * 16, 16)] = _f32(
                        ((tots[u] - 1) << 15) | js[h][u])
            return carry

        lax.fori_loop(0, SCH // 2, _count, jnp.int32(0))

        def _prefix(v, carry):
            xs = [hists[u][pl.ds(v * 16, 16)] for u in range(NSTREAM)]
            s = xs[0]
            for u in range(1, NSTREAM):
                s = s + xs[u]
            base = plsc.cumsum(s) - s + carry
            for u in range(NSTREAM):
                hists[u][pl.ds(v * 16, 16)] = base
                base = base + xs[u]
            return carry + jnp.sum(s)

        lax.fori_loop(0, (mask + 1) // 16, _prefix, jnp.int32(0))

        @plsc.parallel_loop(0, SCH, step=2)
        def _permute(i0, shift=shift, mask=mask):
            iis = [(i0 + h, u) for h in range(2) for u in range(NSTREAM)]
            ws = [_i32(pay_b[pl.ds((i + u * SCH) * 16, 16)])
                  for i, u in iis]
            js = [w & 0x7FFF for w in ws]
            occs = [lax.shift_right_logical(w, 15) for w in ws]
            ks = [_i32(plsc.load_gather(keys, [j])) for j in js]
            bases = [
                plsc.load_gather(
                    hists[iis[x][1]],
                    [lax.shift_right_logical(ks[x], shift) & mask])
                for x in range(2 * NSTREAM)
            ]
            for x in range(2 * NSTREAM):
                plsc.store_scatter(pay_a, [bases[x] + occs[x]], js[x])

    pltpu.sync_copy(km_hbm.at[wid], pay_b)

    @plsc.parallel_loop(0, N // 16, unroll=8)
    def _scatter(i):
        j = pay_a[pl.ds(i * 16, 16)]
        v = pay_b[pl.ds(i * 16, 16)]
        plsc.store_scatter(keys, [j], v)

    pltpu.sync_copy(keys, out_hbm.at[wid])


@jax.jit
def _sc_sort_scatter(km, fn):
    mesh = plsc.VectorSubcoreMesh(core_axis_name="c", subcore_axis_name="s")
    f = pl.kernel(
        _sort_scatter_body,
        out_type=jax.ShapeDtypeStruct((L, N), jnp.float32),
        mesh=mesh,
        compiler_params=pltpu.CompilerParams(needs_layout_passes=False),
        scratch_types=[
            pltpu.VMEM((N,), jnp.float32),
            pltpu.VMEM((N,), jnp.int32),
            pltpu.VMEM((N,), jnp.float32),
        ] + [pltpu.VMEM((NB,), jnp.int32)] * NSTREAM,
    )
    return f(km, fn)


def _sigmoid_body(x_ref, m_ref, o_ref):
    t = (x_ref[...] - m_ref[0, 0]) / 0.7
    o_ref[...] = 1.0 / (1.0 + jnp.exp(-t))


@jax.jit
def _tc_sigmoid(fn, mean):
    return pl.pallas_call(
        _sigmoid_body,
        out_shape=jax.ShapeDtypeStruct((L, N), jnp.float32),
        in_specs=[
            pl.BlockSpec((L, N), lambda: (0, 0)),
            pl.BlockSpec(memory_space=pltpu.SMEM),
        ],
        out_specs=pl.BlockSpec((L, N), lambda: (0, 0)),
    )(fn, jnp.reshape(mean, (1, 1)))


def kernel(k_masks, fn, mean):
    ori_masks = _sc_sort_scatter(k_masks, fn)
    cout = _tc_sigmoid(fn, mean)
    return ori_masks, cout
